# baseline (speedup 1.0000x reference)
import functools

import jax
import jax.numpy as jnp
from jax import lax
from jax.experimental import pallas as pl
from jax.experimental.pallas import tpu as pltpu
from jax.experimental.pallas import tpu_sc as plsc

_NCLS = 1000
_B = 16384
_BLK = 1024

_NPAD = 1024


def _tc_body(yh_ref, y_ref, loss_ref, cls_ref, cnt_ref):
    yh = yh_ref[...]
    yv = y_ref[...]
    m = jnp.max(yh, axis=1, keepdims=True)
    lse = jnp.log(jnp.sum(jnp.exp(yh - m), axis=1, keepdims=True))
    sy = jnp.sum(yv, axis=1)
    syh = jnp.sum(yv * yh, axis=1)
    loss_ref[...] = sy * (m[:, 0] + lse[:, 0]) - syh
    ym = jnp.max(yv, axis=1, keepdims=True)
    colid = lax.broadcasted_iota(jnp.int32, yv.shape, 1)
    cls = jnp.min(jnp.where(yv == ym, colid, _NCLS), axis=1)
    cls_ref[...] = cls
    onehot = (colid == cls[:, None]).astype(jnp.float32)
    partial = jnp.pad(jnp.sum(onehot, axis=0), (0, _NPAD - _NCLS))

    @pl.when(pl.program_id(0) == 0)
    def _():
        cnt_ref[...] = jnp.zeros((_NPAD,), jnp.float32)
    cnt_ref[...] += partial


def _tc_loss(y_hat, y):
    return pl.pallas_call(
        _tc_body,
        grid=(_B // _BLK,),
        in_specs=[
            pl.BlockSpec((_BLK, _NCLS), lambda i: (i, 0)),
            pl.BlockSpec((_BLK, _NCLS), lambda i: (i, 0)),
        ],
        out_specs=[
            pl.BlockSpec((_BLK,), lambda i: (i,)),
            pl.BlockSpec((_BLK,), lambda i: (i,)),
            pl.BlockSpec((_NPAD,), lambda i: (0,)),
        ],
        out_shape=[
            jax.ShapeDtypeStruct((_B,), jnp.float32),
            jax.ShapeDtypeStruct((_B,), jnp.int32),
            jax.ShapeDtypeStruct((_NPAD,), jnp.float32),
        ],
    )(y_hat, y)


def _sc_body(loss_hbm, cls_hbm, cnt_hbm, out_hbm,
             loss_v, cls_v, acc, big_v, cnt_v, out_v, shared):
    c = lax.axis_index("c")
    s = lax.axis_index("s")

    @pl.when(c == 0)
    def _():
        def zero_chunk(i, _):
            acc[pl.ds(i * 16, 16)] = jnp.zeros((16,), jnp.float32)
            return 0
        lax.fori_loop(0, _NPAD // 16, zero_chunk, 0)

        n_per = _B // 16
        base = s * n_per
        pltpu.sync_copy(loss_hbm.at[pl.ds(base, n_per)], loss_v)
        pltpu.sync_copy(cls_hbm.at[pl.ds(base, n_per)], cls_v)

        def accum(j, _):
            lv = loss_v[pl.ds(j * 16, 16)]
            cv = cls_v[pl.ds(j * 16, 16)]
            plsc.addupdate_scatter(acc, [cv], lv)
            return 0
        lax.fori_loop(0, n_per // 16, accum, 0)

        pltpu.sync_copy(acc, shared.at[s])
        plsc.subcore_barrier()
        pltpu.sync_copy(shared, big_v)

        cbase = s * 64
        pltpu.sync_copy(cnt_hbm.at[pl.ds(cbase, 64)], cnt_v)
        for k in range(4):
            def red(t, v):
                return v + big_v[t, pl.ds(cbase + k * 16, 16)]
            vs = lax.fori_loop(0, 16, red, jnp.zeros((16,), jnp.float32))
            out_v[pl.ds(k * 16, 16)] = vs / cnt_v[pl.ds(k * 16, 16)]
        pltpu.sync_copy(out_v, out_hbm.at[pl.ds(cbase, 64)])


def _sc_segment_mean(loss, cls, cnt):
    mesh = plsc.VectorSubcoreMesh(core_axis_name="c", subcore_axis_name="s")
    n_per = _B // 16
    f = functools.partial(
        pl.kernel,
        mesh=mesh,
        out_type=jax.ShapeDtypeStruct((_NPAD,), jnp.float32),
        compiler_params=pltpu.CompilerParams(needs_layout_passes=False),
        scratch_types=[
            pltpu.VMEM((n_per,), jnp.float32),
            pltpu.VMEM((n_per,), jnp.int32),
            pltpu.VMEM((_NPAD,), jnp.float32),
            pltpu.VMEM((16, _NPAD), jnp.float32),
            pltpu.VMEM((64,), jnp.float32),
            pltpu.VMEM((64,), jnp.float32),
            pltpu.VMEM_SHARED((16, _NPAD), jnp.float32),
        ],
    )(_sc_body)
    return f(loss, cls, cnt)


def kernel(y_hat, y):
    loss, cls, cnt = _tc_loss(y_hat, y)
    out = _sc_segment_mean(loss, cls, cnt)
    return out[:_NCLS]


# speedup vs baseline: 1.0355x; 1.0355x over previous
import functools

import jax
import jax.numpy as jnp
from jax import lax
from jax.experimental import pallas as pl
from jax.experimental.pallas import tpu as pltpu
from jax.experimental.pallas import tpu_sc as plsc

_NCLS = 1000
_B = 16384
_BLK = 1024

_NPAD = 1024


def _tc_body(yh_ref, y_ref, loss_ref, cls_ref):
    yh = yh_ref[...]
    yv = y_ref[...]
    m = jnp.max(yh, axis=1, keepdims=True)
    lse = jnp.log(jnp.sum(jnp.exp(yh - m), axis=1, keepdims=True))
    sy = jnp.sum(yv, axis=1)
    syh = jnp.sum(yv * yh, axis=1)
    loss_ref[...] = sy * (m[:, 0] + lse[:, 0]) - syh
    ym = jnp.max(yv, axis=1, keepdims=True)
    colid = lax.broadcasted_iota(jnp.int32, yv.shape, 1)
    cls_ref[...] = jnp.min(jnp.where(yv == ym, colid, _NCLS), axis=1)


def _tc_loss(y_hat, y):
    return pl.pallas_call(
        _tc_body,
        grid=(_B // _BLK,),
        in_specs=[
            pl.BlockSpec((_BLK, _NCLS), lambda i: (i, 0)),
            pl.BlockSpec((_BLK, _NCLS), lambda i: (i, 0)),
        ],
        out_specs=[
            pl.BlockSpec((_BLK,), lambda i: (i,)),
            pl.BlockSpec((_BLK,), lambda i: (i,)),
        ],
        out_shape=[
            jax.ShapeDtypeStruct((_B,), jnp.float32),
            jax.ShapeDtypeStruct((_B,), jnp.int32),
        ],
    )(y_hat, y)


def _sc_body(loss_hbm, cls_hbm, out_hbm,
             loss_v, cls_v, acc, col_v, out_v, shared):
    c = lax.axis_index("c")
    s = lax.axis_index("s")

    @pl.when(c == 0)
    def _():
        def zero_chunk(i, _):
            acc[pl.ds(i * 16, 16)] = jnp.zeros((16,), jnp.float32)
            return 0
        lax.fori_loop(0, 2 * _NPAD // 16, zero_chunk, 0)

        n_per = _B // 16
        base = s * n_per
        pltpu.sync_copy(loss_hbm.at[pl.ds(base, n_per)], loss_v)
        pltpu.sync_copy(cls_hbm.at[pl.ds(base, n_per)], cls_v)

        ones = jnp.ones((16,), jnp.float32)

        def accum(j, _):
            lv = loss_v[pl.ds(j * 16, 16)]
            cv = cls_v[pl.ds(j * 16, 16)]
            plsc.addupdate_scatter(acc, [cv], lv)
            plsc.addupdate_scatter(acc, [cv + _NPAD], ones)
            return 0
        lax.fori_loop(0, n_per // 16, accum, 0)

        pltpu.sync_copy(acc, shared.at[s])
        plsc.subcore_barrier()

        @pl.when(s < 8)
        def _():
            cbase = s * 128
            pltpu.sync_copy(shared.at[:, pl.ds(cbase, 128)],
                            col_v.at[:, 0:128])
            pltpu.sync_copy(shared.at[:, pl.ds(_NPAD + cbase, 128)],
                            col_v.at[:, 128:256])
            for k in range(8):
                def red(t, v):
                    vs, vc = v
                    vs = vs + col_v[t, pl.ds(k * 16, 16)]
                    vc = vc + col_v[t, pl.ds(128 + k * 16, 16)]
                    return (vs, vc)
                z = jnp.zeros((16,), jnp.float32)
                vs, vc = lax.fori_loop(0, 16, red, (z, z))
                out_v[pl.ds(k * 16, 16)] = vs / vc
            pltpu.sync_copy(out_v, out_hbm.at[pl.ds(cbase, 128)])


def _sc_segment_mean(loss, cls):
    mesh = plsc.VectorSubcoreMesh(core_axis_name="c", subcore_axis_name="s")
    n_per = _B // 16
    f = functools.partial(
        pl.kernel,
        mesh=mesh,
        out_type=jax.ShapeDtypeStruct((_NPAD,), jnp.float32),
        compiler_params=pltpu.CompilerParams(needs_layout_passes=False),
        scratch_types=[
            pltpu.VMEM((n_per,), jnp.float32),
            pltpu.VMEM((n_per,), jnp.int32),
            pltpu.VMEM((2 * _NPAD,), jnp.float32),
            pltpu.VMEM((16, 256), jnp.float32),
            pltpu.VMEM((128,), jnp.float32),
            pltpu.VMEM_SHARED((16, 2 * _NPAD), jnp.float32),
        ],
    )(_sc_body)
    return f(loss, cls)


def kernel(y_hat, y):
    loss, cls = _tc_loss(y_hat, y)
    out = _sc_segment_mean(loss, cls)
    return out[:_NCLS]
